# baseline (speedup 1.0000x reference)
import functools

import jax
import jax.numpy as jnp
from jax import lax
from jax.experimental import pallas as pl
from jax.experimental.pallas import tpu as pltpu
from jax.experimental.pallas import tpu_sc as plsc

N = 10000
F = 128
HDIM = 128
G = 64
E = 320000

NC = 2
NS = 16
NW = NC * NS
CHUNK = 128
NBUF = 2
SUPB = 8
EPT = -(-E // NW)
C_TILE = -(-(-(-EPT // CHUNK)) // SUPB) * SUPB
NBODY = C_TILE // SUPB
E_PAD = NW * C_TILE * CHUNK

N_PAD = 10240
RPT = N_PAD // NS

BLK = 512
NB = N_PAD // BLK

_mesh = plsc.VectorSubcoreMesh(core_axis_name="c", subcore_axis_name="s")

CNT_W = 16


@functools.partial(
    pl.kernel,
    mesh=_mesh,
    out_type=jax.ShapeDtypeStruct((2, N_PAD, CNT_W), jnp.float32),
    scratch_types=[
        pltpu.VMEM((C_TILE, CHUNK), jnp.int32),
        pltpu.VMEM((CHUNK, CNT_W), jnp.float32),
        pltpu.VMEM((64, CNT_W), jnp.float32),
        pltpu.VMEM_SHARED((N_PAD, CNT_W), jnp.float32),
    ],
)
def _sc_count(col_hbm, out_hbm, col_v, ones_v, zb_v, cnt_sh):
    cid = lax.axis_index("c")
    sid = lax.axis_index("s")
    wid = sid * NC + cid

    @pl.loop(0, CHUNK)
    def _(i):
        ones_v[i, :] = jnp.ones((CNT_W,), jnp.float32)

    @pl.loop(0, 64)
    def _(i):
        zb_v[i, :] = jnp.zeros((CNT_W,), jnp.float32)

    @pl.loop(0, RPT, step=64)
    def _(r):
        pltpu.sync_copy(zb_v, cnt_sh.at[pl.ds(sid * RPT + r, 64)])

    plsc.subcore_barrier()
    pltpu.sync_copy(col_hbm.at[wid], col_v)

    @pl.loop(0, C_TILE)
    def _(c):
        pltpu.sync_copy(ones_v, cnt_sh.at[col_v.at[c]], add=True)

    plsc.subcore_barrier()
    pltpu.sync_copy(cnt_sh.at[pl.ds(sid * RPT, RPT)],
                    out_hbm.at[cid].at[pl.ds(sid * RPT, RPT)])


@functools.partial(
    pl.kernel,
    mesh=_mesh,
    out_type=jax.ShapeDtypeStruct((2, N_PAD, HDIM), jnp.float32),
    scratch_types=[
        pltpu.VMEM((C_TILE, CHUNK), jnp.int32),
        pltpu.VMEM((C_TILE, CHUNK), jnp.int32),
        pltpu.VMEM((CHUNK, HDIM), jnp.float32),
        pltpu.VMEM((64, HDIM), jnp.float32),
        pltpu.VMEM_SHARED((N_PAD, HDIM), jnp.float32),
        pltpu.SemaphoreType.DMA,
    ],
)
def _sc_agg(h_hbm, row_hbm, col_hbm, out_hbm,
            row_v, col_v, gbuf, zb_v, acc_sh, sem):
    cid = lax.axis_index("c")
    sid = lax.axis_index("s")
    wid = sid * NC + cid

    @pl.loop(0, 64)
    def _(i):
        @pl.loop(0, HDIM, step=16)
        def _(j):
            zb_v[i, pl.ds(j, 16)] = jnp.zeros((16,), jnp.float32)

    @pl.loop(0, RPT, step=64)
    def _(r):
        pltpu.sync_copy(zb_v, acc_sh.at[pl.ds(sid * RPT + r, 64)])

    plsc.subcore_barrier()
    pltpu.sync_copy(row_hbm.at[wid], row_v)
    pltpu.sync_copy(col_hbm.at[wid], col_v)

    @pl.loop(0, C_TILE)
    def _(c):
        pltpu.async_copy(h_hbm.at[row_v.at[c]], gbuf, sem).wait()
        pltpu.sync_copy(gbuf, acc_sh.at[col_v.at[c]], add=True)

    plsc.subcore_barrier()
    pltpu.sync_copy(acc_sh.at[pl.ds(sid * RPT, RPT)],
                    out_hbm.at[cid].at[pl.ds(sid * RPT, RPT)])


def _prep_body(cnt_ref, x_ref, w_ref, dinv_ref, hp_ref):
    deg = cnt_ref[0, :, 0:1] + cnt_ref[1, :, 0:1] + 1.0
    dinv = lax.rsqrt(deg)
    dinv_ref[...] = dinv
    h = jnp.dot(x_ref[...], w_ref[...], preferred_element_type=jnp.float32)
    hp_ref[...] = h * dinv


def _tc_prep(cnt, x_p, W1):
    return pl.pallas_call(
        _prep_body,
        grid=(NB,),
        in_specs=[
            pl.BlockSpec((2, BLK, CNT_W), lambda i: (0, i, 0)),
            pl.BlockSpec((BLK, F), lambda i: (i, 0)),
            pl.BlockSpec((F, HDIM), lambda i: (0, 0)),
        ],
        out_specs=[
            pl.BlockSpec((BLK, 1), lambda i: (i, 0)),
            pl.BlockSpec((BLK, HDIM), lambda i: (i, 0)),
        ],
        out_shape=[
            jax.ShapeDtypeStruct((N_PAD, 1), jnp.float32),
            jax.ShapeDtypeStruct((N_PAD, HDIM), jnp.float32),
        ],
    )(cnt, x_p, W1)


def _layer_body(agg_ref, hp_ref, dinv_ref, b_ref, w_ref, out_ref):
    dinv = dinv_ref[...]
    a = agg_ref[0] + agg_ref[1] + hp_ref[...]
    xn = jnp.maximum(dinv * a + b_ref[...], 0.0)
    h = jnp.dot(xn, w_ref[...], preferred_element_type=jnp.float32)
    out_ref[...] = h * dinv


def _tc_layer(agg, hp, dinv, b2d, Wn):
    return pl.pallas_call(
        _layer_body,
        grid=(NB,),
        in_specs=[
            pl.BlockSpec((2, BLK, HDIM), lambda i: (0, i, 0)),
            pl.BlockSpec((BLK, HDIM), lambda i: (i, 0)),
            pl.BlockSpec((BLK, 1), lambda i: (i, 0)),
            pl.BlockSpec((1, HDIM), lambda i: (0, 0)),
            pl.BlockSpec((HDIM, HDIM), lambda i: (0, 0)),
        ],
        out_specs=pl.BlockSpec((BLK, HDIM), lambda i: (i, 0)),
        out_shape=jax.ShapeDtypeStruct((N_PAD, HDIM), jnp.float32),
    )(agg, hp, dinv, b2d, Wn)


def _final_body(agg_ref, hp_ref, dinv_ref, b_ref, batch_ref, wl_ref, bl_ref,
                out_ref, pooled_acc, cnt_acc):
    i = pl.program_id(0)

    @pl.when(i == 0)
    def _():
        pooled_acc[...] = jnp.zeros_like(pooled_acc)
        cnt_acc[...] = jnp.zeros_like(cnt_acc)

    h3 = dinv_ref[...] * (agg_ref[0] + agg_ref[1] + hp_ref[...]) + b_ref[...]
    bvec = batch_ref[0, 0, :]
    gids = lax.broadcasted_iota(jnp.int32, (G, BLK), 0)
    mask = (bvec[None, :] == gids).astype(jnp.float32)
    pooled_acc[...] += jnp.dot(mask, h3, preferred_element_type=jnp.float32)
    cnt_acc[...] = cnt_acc[...] + jnp.sum(mask, axis=1, keepdims=True)

    @pl.when(i == NB - 1)
    def _():
        pooled = pooled_acc[...] / jnp.maximum(cnt_acc[...], 1.0)
        out_ref[...] = (
            jnp.dot(pooled, wl_ref[...], preferred_element_type=jnp.float32)
            + bl_ref[...]
        )


def _tc_final(agg, hp, dinv, b2d, batch3d, Wl, bl2d):
    return pl.pallas_call(
        _final_body,
        grid=(NB,),
        in_specs=[
            pl.BlockSpec((2, BLK, HDIM), lambda i: (0, i, 0)),
            pl.BlockSpec((BLK, HDIM), lambda i: (i, 0)),
            pl.BlockSpec((BLK, 1), lambda i: (i, 0)),
            pl.BlockSpec((1, HDIM), lambda i: (0, 0)),
            pl.BlockSpec((1, 1, BLK), lambda i: (i, 0, 0)),
            pl.BlockSpec((HDIM, 1), lambda i: (0, 0)),
            pl.BlockSpec((1, 1), lambda i: (0, 0)),
        ],
        out_specs=pl.BlockSpec((G, 1), lambda i: (0, 0)),
        out_shape=jax.ShapeDtypeStruct((G, 1), jnp.float32),
        scratch_shapes=[
            pltpu.VMEM((G, HDIM), jnp.float32),
            pltpu.VMEM((G, HDIM), jnp.float32),
        ],
    )(agg, hp, dinv, b2d, batch3d, Wl, bl2d)


def kernel(x, edge_index, batch, W1, b1, W2, b2, W3, b3, Wl, bl):
    row = edge_index[0]
    col = edge_index[1]
    row_p = jnp.concatenate(
        [row, jnp.zeros((E_PAD - E,), jnp.int32)]).reshape(NW, C_TILE, CHUNK)
    col_p = jnp.concatenate(
        [col, jnp.full((E_PAD - E,), N, jnp.int32)]).reshape(NW, C_TILE, CHUNK)
    x_p = jnp.pad(x, ((0, N_PAD - N), (0, 0)))
    batch3d = jnp.concatenate(
        [batch, jnp.full((N_PAD - N,), G, jnp.int32)]).reshape(NB, 1, BLK)
    b1d = b1.reshape(1, HDIM)
    b2d = b2.reshape(1, HDIM)
    b3d = b3.reshape(1, HDIM)
    bl2d = bl.reshape(1, 1)

    cnt = _sc_count(col_p)
    dinv, h1p = _tc_prep(cnt, x_p, W1)
    a1 = _sc_agg(h1p, row_p, col_p)
    h2p = _tc_layer(a1, h1p, dinv, b1d, W2)
    a2 = _sc_agg(h2p, row_p, col_p)
    h3p = _tc_layer(a2, h2p, dinv, b2d, W3)
    a3 = _sc_agg(h3p, row_p, col_p)
    return _tc_final(a3, h3p, dinv, b3d, batch3d, Wl, bl2d)


# speedup vs baseline: 1.4612x; 1.4612x over previous
import functools

import jax
import jax.numpy as jnp
from jax import lax
from jax.experimental import pallas as pl
from jax.experimental.pallas import tpu as pltpu
from jax.experimental.pallas import tpu_sc as plsc

N = 10000
F = 128
HDIM = 128
G = 64
E = 320000

NC = 2
NS = 16
NW = NC * NS
CHUNK = 128
NBUF = 2
SUPB = 8
EPT = -(-E // NW)
C_TILE = -(-EPT // CHUNK)
E_PAD = NW * C_TILE * CHUNK

N_PAD = 10240
RPT = N_PAD // NS

BLK = 512
NB = N_PAD // BLK

_mesh = plsc.VectorSubcoreMesh(core_axis_name="c", subcore_axis_name="s")

CNT_W = 16


@functools.partial(
    pl.kernel,
    mesh=_mesh,
    out_type=jax.ShapeDtypeStruct((2, N_PAD, CNT_W), jnp.float32),
    scratch_types=[
        pltpu.VMEM((C_TILE, CHUNK), jnp.int32),
        pltpu.VMEM((CHUNK, CNT_W), jnp.float32),
        pltpu.VMEM((64, CNT_W), jnp.float32),
        pltpu.VMEM_SHARED((N_PAD, CNT_W), jnp.float32),
    ],
)
def _sc_count(col_hbm, out_hbm, col_v, ones_v, zb_v, cnt_sh):
    cid = lax.axis_index("c")
    sid = lax.axis_index("s")
    wid = sid * NC + cid

    @pl.loop(0, CHUNK)
    def _(i):
        ones_v[i, :] = jnp.ones((CNT_W,), jnp.float32)

    @pl.loop(0, 64)
    def _(i):
        zb_v[i, :] = jnp.zeros((CNT_W,), jnp.float32)

    @pl.loop(0, RPT, step=64)
    def _(r):
        pltpu.sync_copy(zb_v, cnt_sh.at[pl.ds(sid * RPT + r, 64)])

    plsc.subcore_barrier()
    pltpu.sync_copy(col_hbm.at[wid], col_v)

    @pl.loop(0, C_TILE)
    def _(c):
        pltpu.sync_copy(ones_v, cnt_sh.at[col_v.at[c]], add=True)

    plsc.subcore_barrier()
    pltpu.sync_copy(cnt_sh.at[pl.ds(sid * RPT, RPT)],
                    out_hbm.at[cid].at[pl.ds(sid * RPT, RPT)])


@functools.partial(
    pl.kernel,
    mesh=_mesh,
    out_type=jax.ShapeDtypeStruct((2, N_PAD, HDIM), jnp.float32),
    scratch_types=[
        pltpu.VMEM((C_TILE, CHUNK), jnp.int32),
        pltpu.VMEM((C_TILE, CHUNK), jnp.int32),
        pltpu.VMEM((CHUNK, HDIM), jnp.float32),
        pltpu.VMEM((64, HDIM), jnp.float32),
        pltpu.VMEM_SHARED((N_PAD, HDIM), jnp.float32),
        pltpu.SemaphoreType.DMA,
    ],
)
def _sc_agg(h_hbm, row_hbm, col_hbm, out_hbm,
            row_v, col_v, gbuf, zb_v, acc_sh, sem):
    cid = lax.axis_index("c")
    sid = lax.axis_index("s")
    wid = sid * NC + cid

    @pl.loop(0, 64)
    def _(i):
        @pl.loop(0, HDIM, step=16)
        def _(j):
            zb_v[i, pl.ds(j, 16)] = jnp.zeros((16,), jnp.float32)

    @pl.loop(0, RPT, step=64)
    def _(r):
        pltpu.sync_copy(zb_v, acc_sh.at[pl.ds(sid * RPT + r, 64)])

    plsc.subcore_barrier()
    pltpu.sync_copy(row_hbm.at[wid], row_v)
    pltpu.sync_copy(col_hbm.at[wid], col_v)

    @pl.loop(0, C_TILE)
    def _(c):
        pltpu.async_copy(h_hbm.at[row_v.at[c]], gbuf, sem).wait()
        pltpu.sync_copy(gbuf, acc_sh.at[col_v.at[c]], add=True)

    plsc.subcore_barrier()
    pltpu.sync_copy(acc_sh.at[pl.ds(sid * RPT, RPT)],
                    out_hbm.at[cid].at[pl.ds(sid * RPT, RPT)])


def _prep_body(cnt_ref, x_ref, w_ref, dinv_ref, hp_ref):
    deg = cnt_ref[0, :, 0:1] + cnt_ref[1, :, 0:1] + 1.0
    dinv = lax.rsqrt(deg)
    dinv_ref[...] = dinv
    h = jnp.dot(x_ref[...], w_ref[...], preferred_element_type=jnp.float32)
    hp_ref[...] = h * dinv


def _tc_prep(cnt, x_p, W1):
    return pl.pallas_call(
        _prep_body,
        grid=(NB,),
        in_specs=[
            pl.BlockSpec((2, BLK, CNT_W), lambda i: (0, i, 0)),
            pl.BlockSpec((BLK, F), lambda i: (i, 0)),
            pl.BlockSpec((F, HDIM), lambda i: (0, 0)),
        ],
        out_specs=[
            pl.BlockSpec((BLK, 1), lambda i: (i, 0)),
            pl.BlockSpec((BLK, HDIM), lambda i: (i, 0)),
        ],
        out_shape=[
            jax.ShapeDtypeStruct((N_PAD, 1), jnp.float32),
            jax.ShapeDtypeStruct((N_PAD, HDIM), jnp.float32),
        ],
    )(cnt, x_p, W1)


def _layer_body(agg_ref, hp_ref, dinv_ref, b_ref, w_ref, out_ref):
    dinv = dinv_ref[...]
    a = agg_ref[0] + agg_ref[1] + hp_ref[...]
    xn = jnp.maximum(dinv * a + b_ref[...], 0.0)
    h = jnp.dot(xn, w_ref[...], preferred_element_type=jnp.float32)
    out_ref[...] = h * dinv


def _tc_layer(agg, hp, dinv, b2d, Wn):
    return pl.pallas_call(
        _layer_body,
        grid=(NB,),
        in_specs=[
            pl.BlockSpec((2, BLK, HDIM), lambda i: (0, i, 0)),
            pl.BlockSpec((BLK, HDIM), lambda i: (i, 0)),
            pl.BlockSpec((BLK, 1), lambda i: (i, 0)),
            pl.BlockSpec((1, HDIM), lambda i: (0, 0)),
            pl.BlockSpec((HDIM, HDIM), lambda i: (0, 0)),
        ],
        out_specs=pl.BlockSpec((BLK, HDIM), lambda i: (i, 0)),
        out_shape=jax.ShapeDtypeStruct((N_PAD, HDIM), jnp.float32),
    )(agg, hp, dinv, b2d, Wn)


def _final_body(agg_ref, hp_ref, dinv_ref, b_ref, batch_ref, wl_ref, bl_ref,
                out_ref, pooled_acc, cnt_acc):
    i = pl.program_id(0)

    @pl.when(i == 0)
    def _():
        pooled_acc[...] = jnp.zeros_like(pooled_acc)
        cnt_acc[...] = jnp.zeros_like(cnt_acc)

    h3 = dinv_ref[...] * (agg_ref[0] + agg_ref[1] + hp_ref[...]) + b_ref[...]
    bvec = batch_ref[0, 0, :]
    gids = lax.broadcasted_iota(jnp.int32, (G, BLK), 0)
    mask = (bvec[None, :] == gids).astype(jnp.float32)
    pooled_acc[...] += jnp.dot(mask, h3, preferred_element_type=jnp.float32)
    cnt_acc[...] = cnt_acc[...] + jnp.sum(mask, axis=1, keepdims=True)

    @pl.when(i == NB - 1)
    def _():
        pooled = pooled_acc[...] / jnp.maximum(cnt_acc[...], 1.0)
        out_ref[...] = (
            jnp.dot(pooled, wl_ref[...], preferred_element_type=jnp.float32)
            + bl_ref[...]
        )


def _tc_final(agg, hp, dinv, b2d, batch3d, Wl, bl2d):
    return pl.pallas_call(
        _final_body,
        grid=(NB,),
        in_specs=[
            pl.BlockSpec((2, BLK, HDIM), lambda i: (0, i, 0)),
            pl.BlockSpec((BLK, HDIM), lambda i: (i, 0)),
            pl.BlockSpec((BLK, 1), lambda i: (i, 0)),
            pl.BlockSpec((1, HDIM), lambda i: (0, 0)),
            pl.BlockSpec((1, 1, BLK), lambda i: (i, 0, 0)),
            pl.BlockSpec((HDIM, 1), lambda i: (0, 0)),
            pl.BlockSpec((1, 1), lambda i: (0, 0)),
        ],
        out_specs=pl.BlockSpec((G, 1), lambda i: (0, 0)),
        out_shape=jax.ShapeDtypeStruct((G, 1), jnp.float32),
        scratch_shapes=[
            pltpu.VMEM((G, HDIM), jnp.float32),
            pltpu.VMEM((G, HDIM), jnp.float32),
        ],
    )(agg, hp, dinv, b2d, batch3d, Wl, bl2d)


def kernel(x, edge_index, batch, W1, b1, W2, b2, W3, b3, Wl, bl):
    row = edge_index[0]
    col = edge_index[1]
    row_p = jnp.concatenate(
        [row, jnp.zeros((E_PAD - E,), jnp.int32)]).reshape(NW, C_TILE, CHUNK)
    col_p = jnp.concatenate(
        [col, jnp.full((E_PAD - E,), N, jnp.int32)]).reshape(NW, C_TILE, CHUNK)
    x_p = jnp.pad(x, ((0, N_PAD - N), (0, 0)))
    batch3d = jnp.concatenate(
        [batch, jnp.full((N_PAD - N,), G, jnp.int32)]).reshape(NB, 1, BLK)
    b1d = b1.reshape(1, HDIM)
    b2d = b2.reshape(1, HDIM)
    b3d = b3.reshape(1, HDIM)
    bl2d = bl.reshape(1, 1)

    cnt = _sc_count(col_p)
    dinv, h1p = _tc_prep(cnt, x_p, W1)
    a1 = _sc_agg(h1p, row_p, col_p)
    h2p = _tc_layer(a1, h1p, dinv, b1d, W2)
    a2 = _sc_agg(h2p, row_p, col_p)
    h3p = _tc_layer(a2, h2p, dinv, b2d, W3)
    a3 = _sc_agg(h3p, row_p, col_p)
    return _tc_final(a3, h3p, dinv, b3d, batch3d, Wl, bl2d)


# speedup vs baseline: 1.4643x; 1.0021x over previous
import functools

import jax
import jax.numpy as jnp
from jax import lax
from jax.experimental import pallas as pl
from jax.experimental.pallas import tpu as pltpu
from jax.experimental.pallas import tpu_sc as plsc

N = 10000
F = 128
HDIM = 128
G = 64
E = 320000

NC = 2
NS = 16
NW = NC * NS
CHUNK = 128
EPT = -(-E // NW)
C_TILE = -(-EPT // CHUNK)
E_PAD = NW * C_TILE * CHUNK

N_PAD = 10240
RPT = N_PAD // NS

BLK = 512
NB = N_PAD // BLK

_mesh = plsc.VectorSubcoreMesh(core_axis_name="c", subcore_axis_name="s")

CNT_W = 16


@functools.partial(
    pl.kernel,
    mesh=_mesh,
    out_type=jax.ShapeDtypeStruct((2, N_PAD, CNT_W), jnp.float32),
    scratch_types=[
        pltpu.VMEM((C_TILE, CHUNK), jnp.int32),
        pltpu.VMEM((CHUNK, CNT_W), jnp.float32),
        pltpu.VMEM((64, CNT_W), jnp.float32),
        pltpu.VMEM_SHARED((N_PAD, CNT_W), jnp.float32),
    ],
)
def _sc_count(col_hbm, out_hbm, col_v, ones_v, zb_v, cnt_sh):
    cid = lax.axis_index("c")
    sid = lax.axis_index("s")
    wid = sid * NC + cid

    @pl.loop(0, CHUNK)
    def _(i):
        ones_v[i, :] = jnp.ones((CNT_W,), jnp.float32)

    @pl.loop(0, 64)
    def _(i):
        zb_v[i, :] = jnp.zeros((CNT_W,), jnp.float32)

    @pl.loop(0, RPT, step=64)
    def _(r):
        pltpu.sync_copy(zb_v, cnt_sh.at[pl.ds(sid * RPT + r, 64)])

    plsc.subcore_barrier()
    pltpu.sync_copy(col_hbm.at[wid], col_v)

    @pl.loop(0, C_TILE)
    def _(c):
        pltpu.sync_copy(ones_v, cnt_sh.at[col_v.at[c]], add=True)

    plsc.subcore_barrier()
    pltpu.sync_copy(cnt_sh.at[pl.ds(sid * RPT, RPT)],
                    out_hbm.at[cid].at[pl.ds(sid * RPT, RPT)])


@functools.partial(
    pl.kernel,
    mesh=_mesh,
    out_type=jax.ShapeDtypeStruct((2, N_PAD, HDIM), jnp.float32),
    scratch_types=[
        pltpu.VMEM((C_TILE, CHUNK), jnp.int32),
        pltpu.VMEM((C_TILE, CHUNK), jnp.int32),
        pltpu.VMEM((CHUNK, HDIM), jnp.float32),
        pltpu.VMEM((64, HDIM), jnp.float32),
        pltpu.VMEM_SHARED((N_PAD, HDIM), jnp.float32),
        pltpu.SemaphoreType.DMA,
    ],
)
def _sc_agg(h_hbm, row_hbm, col_hbm, out_hbm,
            row_v, col_v, gbuf, zb_v, acc_sh, sem):
    cid = lax.axis_index("c")
    sid = lax.axis_index("s")
    wid = sid * NC + cid

    @pl.loop(0, 64)
    def _(i):
        @pl.loop(0, HDIM, step=16)
        def _(j):
            zb_v[i, pl.ds(j, 16)] = jnp.zeros((16,), jnp.float32)

    @pl.loop(0, RPT, step=64)
    def _(r):
        pltpu.sync_copy(zb_v, acc_sh.at[pl.ds(sid * RPT + r, 64)])

    plsc.subcore_barrier()
    pltpu.sync_copy(row_hbm.at[wid], row_v)
    pltpu.sync_copy(col_hbm.at[wid], col_v)

    @pl.loop(0, C_TILE)
    def _(c):
        pltpu.async_copy(h_hbm.at[row_v.at[c]], gbuf, sem).wait()
        pltpu.sync_copy(gbuf, acc_sh.at[col_v.at[c]], add=True)

    plsc.subcore_barrier()
    pltpu.sync_copy(acc_sh.at[pl.ds(sid * RPT, RPT)],
                    out_hbm.at[cid].at[pl.ds(sid * RPT, RPT)])


def _prep_body(cnt_ref, x_ref, w_ref, dinv_ref, hp_ref):
    deg = cnt_ref[0, :, 0:1] + cnt_ref[1, :, 0:1] + 1.0
    dinv = lax.rsqrt(deg)
    dinv_ref[...] = dinv
    h = jnp.dot(x_ref[...], w_ref[...], preferred_element_type=jnp.float32)
    hp_ref[...] = h * dinv


def _tc_prep(cnt, x_p, W1):
    return pl.pallas_call(
        _prep_body,
        grid=(NB,),
        in_specs=[
            pl.BlockSpec((2, BLK, CNT_W), lambda i: (0, i, 0)),
            pl.BlockSpec((BLK, F), lambda i: (i, 0)),
            pl.BlockSpec((F, HDIM), lambda i: (0, 0)),
        ],
        out_specs=[
            pl.BlockSpec((BLK, 1), lambda i: (i, 0)),
            pl.BlockSpec((BLK, HDIM), lambda i: (i, 0)),
        ],
        out_shape=[
            jax.ShapeDtypeStruct((N_PAD, 1), jnp.float32),
            jax.ShapeDtypeStruct((N_PAD, HDIM), jnp.float32),
        ],
    )(cnt, x_p, W1)


def _layer_body(agg_ref, hp_ref, dinv_ref, b_ref, w_ref, out_ref):
    dinv = dinv_ref[...]
    a = agg_ref[0] + agg_ref[1] + hp_ref[...]
    xn = jnp.maximum(dinv * a + b_ref[...], 0.0)
    h = jnp.dot(xn, w_ref[...], preferred_element_type=jnp.float32)
    out_ref[...] = h * dinv


def _tc_layer(agg, hp, dinv, b2d, Wn):
    return pl.pallas_call(
        _layer_body,
        grid=(NB,),
        in_specs=[
            pl.BlockSpec((2, BLK, HDIM), lambda i: (0, i, 0)),
            pl.BlockSpec((BLK, HDIM), lambda i: (i, 0)),
            pl.BlockSpec((BLK, 1), lambda i: (i, 0)),
            pl.BlockSpec((1, HDIM), lambda i: (0, 0)),
            pl.BlockSpec((HDIM, HDIM), lambda i: (0, 0)),
        ],
        out_specs=pl.BlockSpec((BLK, HDIM), lambda i: (i, 0)),
        out_shape=jax.ShapeDtypeStruct((N_PAD, HDIM), jnp.float32),
    )(agg, hp, dinv, b2d, Wn)


def _final_body(agg_ref, hp_ref, dinv_ref, b_ref, batch_ref, wl_ref, bl_ref,
                out_ref, pooled_acc, cnt_acc):
    i = pl.program_id(0)

    @pl.when(i == 0)
    def _():
        pooled_acc[...] = jnp.zeros_like(pooled_acc)
        cnt_acc[...] = jnp.zeros_like(cnt_acc)

    h3 = dinv_ref[...] * (agg_ref[0] + agg_ref[1] + hp_ref[...]) + b_ref[...]
    bvec = batch_ref[0, 0, :]
    gids = lax.broadcasted_iota(jnp.int32, (G, BLK), 0)
    mask = (bvec[None, :] == gids).astype(jnp.float32)
    pooled_acc[...] += jnp.dot(mask, h3, preferred_element_type=jnp.float32)
    cnt_acc[...] = cnt_acc[...] + jnp.sum(mask, axis=1, keepdims=True)

    @pl.when(i == NB - 1)
    def _():
        pooled = pooled_acc[...] / jnp.maximum(cnt_acc[...], 1.0)
        out_ref[...] = (
            jnp.dot(pooled, wl_ref[...], preferred_element_type=jnp.float32)
            + bl_ref[...]
        )


def _tc_final(agg, hp, dinv, b2d, batch3d, Wl, bl2d):
    return pl.pallas_call(
        _final_body,
        grid=(NB,),
        in_specs=[
            pl.BlockSpec((2, BLK, HDIM), lambda i: (0, i, 0)),
            pl.BlockSpec((BLK, HDIM), lambda i: (i, 0)),
            pl.BlockSpec((BLK, 1), lambda i: (i, 0)),
            pl.BlockSpec((1, HDIM), lambda i: (0, 0)),
            pl.BlockSpec((1, 1, BLK), lambda i: (i, 0, 0)),
            pl.BlockSpec((HDIM, 1), lambda i: (0, 0)),
            pl.BlockSpec((1, 1), lambda i: (0, 0)),
        ],
        out_specs=pl.BlockSpec((G, 1), lambda i: (0, 0)),
        out_shape=jax.ShapeDtypeStruct((G, 1), jnp.float32),
        scratch_shapes=[
            pltpu.VMEM((G, HDIM), jnp.float32),
            pltpu.VMEM((G, HDIM), jnp.float32),
        ],
    )(agg, hp, dinv, b2d, batch3d, Wl, bl2d)


def kernel(x, edge_index, batch, W1, b1, W2, b2, W3, b3, Wl, bl):
    row = edge_index[0]
    col = edge_index[1]
    row_p = jnp.concatenate(
        [row, jnp.zeros((E_PAD - E,), jnp.int32)]).reshape(NW, C_TILE, CHUNK)
    col_p = jnp.concatenate(
        [col, jnp.full((E_PAD - E,), N, jnp.int32)]).reshape(NW, C_TILE, CHUNK)
    x_p = jnp.pad(x, ((0, N_PAD - N), (0, 0)))
    batch3d = jnp.concatenate(
        [batch, jnp.full((N_PAD - N,), G, jnp.int32)]).reshape(NB, 1, BLK)
    b1d = b1.reshape(1, HDIM)
    b2d = b2.reshape(1, HDIM)
    b3d = b3.reshape(1, HDIM)
    bl2d = bl.reshape(1, 1)

    cnt = _sc_count(col_p)
    dinv, h1p = _tc_prep(cnt, x_p, W1)
    a1 = _sc_agg(h1p, row_p, col_p)
    h2p = _tc_layer(a1, h1p, dinv, b1d, W2)
    a2 = _sc_agg(h2p, row_p, col_p)
    h3p = _tc_layer(a2, h2p, dinv, b2d, W3)
    a3 = _sc_agg(h3p, row_p, col_p)
    return _tc_final(a3, h3p, dinv, b3d, batch3d, Wl, bl2d)
